# baseline (speedup 1.0000x reference)
import functools

import jax
import jax.numpy as jnp
from jax.experimental import pallas as pl
from jax.experimental.pallas import tpu as pltpu

TOKENS = 32768
EMBED_DIM = 768
NUM_EXPERTS = 8
TOP_K = 2
BLOCK = 2048


def _router_kernel(x_ref, w_ref, idx_ref, wgt_ref, logits_ref):
    x = x_ref[...]
    w = w_ref[...]
    logits = jax.lax.dot_general(
        x, w, (((1,), (1,)), ((), ())), preferred_element_type=jnp.float32
    )
    logits_ref[...] = logits

    i1 = jnp.argmax(logits, axis=-1)
    m1 = jnp.max(logits, axis=-1)
    e = jax.lax.broadcasted_iota(jnp.int32, logits.shape, 1)
    masked = jnp.where(e == i1[:, None], -jnp.inf, logits)
    i2 = jnp.argmax(masked, axis=-1)
    m2 = jnp.max(masked, axis=-1)

    w1 = 1.0 / (1.0 + jnp.exp(m2 - m1))
    w2 = 1.0 - w1

    idx_ref[...] = jnp.concatenate(
        [i1[:, None].astype(jnp.int32), i2[:, None].astype(jnp.int32)], axis=1
    )
    wgt_ref[...] = jnp.concatenate([w1[:, None], w2[:, None]], axis=1)


@jax.jit
def kernel(x, W):
    nb = TOKENS // BLOCK
    idx, wgt, logits = pl.pallas_call(
        _router_kernel,
        grid=(nb,),
        in_specs=[
            pl.BlockSpec((BLOCK, EMBED_DIM), lambda i: (i, 0)),
            pl.BlockSpec((NUM_EXPERTS, EMBED_DIM), lambda i: (0, 0)),
        ],
        out_specs=[
            pl.BlockSpec((BLOCK, TOP_K), lambda i: (i, 0)),
            pl.BlockSpec((BLOCK, TOP_K), lambda i: (i, 0)),
            pl.BlockSpec((BLOCK, NUM_EXPERTS), lambda i: (i, 0)),
        ],
        compiler_params=pltpu.CompilerParams(
            dimension_semantics=("parallel",),
        ),
        out_shape=[
            jax.ShapeDtypeStruct((TOKENS, TOP_K), jnp.int32),
            jax.ShapeDtypeStruct((TOKENS, TOP_K), jnp.float32),
            jax.ShapeDtypeStruct((TOKENS, NUM_EXPERTS), jnp.float32),
        ],
    )(x, W)
    return idx, wgt, logits


# speedup vs baseline: 2.3763x; 2.3763x over previous
import functools

import jax
import jax.numpy as jnp
from jax.experimental import pallas as pl
from jax.experimental.pallas import tpu as pltpu

TOKENS = 32768
EMBED_DIM = 768
NUM_EXPERTS = 8
TOP_K = 2
BLOCK = 4096


def _router_kernel(x_ref, w_ref, idx_ref, wgt_ref, logits_ref):
    x = x_ref[...]
    w = w_ref[...]
    logits = jax.lax.dot_general(
        x, w, (((1,), (1,)), ((), ())), preferred_element_type=jnp.float32
    )
    logits_t = logits.T
    logits_ref[...] = logits_t

    i1 = jnp.argmax(logits_t, axis=0)
    m1 = jnp.max(logits_t, axis=0)
    e = jax.lax.broadcasted_iota(jnp.int32, logits_t.shape, 0)
    masked = jnp.where(e == i1[None, :], -jnp.inf, logits_t)
    i2 = jnp.argmax(masked, axis=0)
    m2 = jnp.max(masked, axis=0)

    w1 = 1.0 / (1.0 + jnp.exp(m2 - m1))
    w2 = 1.0 - w1

    idx_ref[...] = jnp.concatenate(
        [i1[None, :].astype(jnp.int32), i2[None, :].astype(jnp.int32)], axis=0
    )
    wgt_ref[...] = jnp.concatenate([w1[None, :], w2[None, :]], axis=0)


@jax.jit
def kernel(x, W):
    nb = TOKENS // BLOCK
    idx_t, wgt_t, logits_t = pl.pallas_call(
        _router_kernel,
        grid=(nb,),
        in_specs=[
            pl.BlockSpec((BLOCK, EMBED_DIM), lambda i: (i, 0)),
            pl.BlockSpec((NUM_EXPERTS, EMBED_DIM), lambda i: (0, 0)),
        ],
        out_specs=[
            pl.BlockSpec((TOP_K, BLOCK), lambda i: (0, i)),
            pl.BlockSpec((TOP_K, BLOCK), lambda i: (0, i)),
            pl.BlockSpec((NUM_EXPERTS, BLOCK), lambda i: (0, i)),
        ],
        compiler_params=pltpu.CompilerParams(
            dimension_semantics=("arbitrary",),
        ),
        out_shape=[
            jax.ShapeDtypeStruct((TOP_K, TOKENS), jnp.int32),
            jax.ShapeDtypeStruct((TOP_K, TOKENS), jnp.float32),
            jax.ShapeDtypeStruct((NUM_EXPERTS, TOKENS), jnp.float32),
        ],
    )(x, W)
    return idx_t.T, wgt_t.T, logits_t.T


# speedup vs baseline: 2.4217x; 1.0191x over previous
import functools

import jax
import jax.numpy as jnp
from jax.experimental import pallas as pl
from jax.experimental.pallas import tpu as pltpu

TOKENS = 32768
EMBED_DIM = 768
NUM_EXPERTS = 8
TOP_K = 2
BLOCK = 4096


def _router_kernel(x_ref, w_ref, idx_ref, wgt_ref, logits_ref):
    x = x_ref[...]
    w = w_ref[...]
    logits = jax.lax.dot_general(
        x, w, (((1,), (1,)), ((), ())), preferred_element_type=jnp.float32
    )
    logits_t = logits.T
    logits_ref[...] = logits_t

    i1 = jnp.argmax(logits_t, axis=0)
    m1 = jnp.max(logits_t, axis=0)
    e = jax.lax.broadcasted_iota(jnp.int32, logits_t.shape, 0)
    masked = jnp.where(e == i1[None, :], -jnp.inf, logits_t)
    i2 = jnp.argmax(masked, axis=0)
    m2 = jnp.max(masked, axis=0)

    w1 = 1.0 / (1.0 + jnp.exp(m2 - m1))
    w2 = 1.0 - w1

    idx_ref[...] = jnp.concatenate(
        [i1[None, :].astype(jnp.int32), i2[None, :].astype(jnp.int32)], axis=0
    )
    wgt_ref[...] = jnp.concatenate([w1[None, :], w2[None, :]], axis=0)


@jax.jit
def kernel(x, W):
    nb = TOKENS // BLOCK
    idx_t, wgt_t, logits_t = pl.pallas_call(
        _router_kernel,
        grid=(nb,),
        in_specs=[
            pl.BlockSpec((BLOCK, EMBED_DIM), lambda i: (i, 0)),
            pl.BlockSpec((NUM_EXPERTS, EMBED_DIM), lambda i: (0, 0)),
        ],
        out_specs=[
            pl.BlockSpec((TOP_K, BLOCK), lambda i: (0, i)),
            pl.BlockSpec((TOP_K, BLOCK), lambda i: (0, i)),
            pl.BlockSpec((NUM_EXPERTS, BLOCK), lambda i: (0, i)),
        ],
        compiler_params=pltpu.CompilerParams(
            dimension_semantics=("parallel",),
        ),
        out_shape=[
            jax.ShapeDtypeStruct((TOP_K, TOKENS), jnp.int32),
            jax.ShapeDtypeStruct((TOP_K, TOKENS), jnp.float32),
            jax.ShapeDtypeStruct((NUM_EXPERTS, TOKENS), jnp.float32),
        ],
    )(x, W)
    return idx_t.T, wgt_t.T, logits_t.T
